# Initial kernel scaffold; baseline (speedup 1.0000x reference)
import jax
import jax.numpy as jnp
from jax.experimental import pallas as pl


def kernel(x, freqs_cos, freqs_sin, norm1_w, norm2_w, Wq, Wk, Wv, Wo, gate_w, Weg, Weu, Wed, Wsg, Wsu, Wsd):
    raise NotImplementedError("write your pallas kernel here")


# speedup vs baseline: 1.4924x; 1.4924x over previous
import math

import jax
import jax.numpy as jnp
from jax.experimental import pallas as pl
from jax.experimental.pallas import tpu as pltpu

EPS = 1e-6
NEG = -1e30

_C11 = (((1,), (1,)), ((), ()))
_C10 = (((1,), (0,)), ((), ()))


def _dot(a, b, dn=_C11):
    return jax.lax.dot_general(a, b, dn, preferred_element_type=jnp.float32)


def _qkv_body(nh, dh, x_ref, n1_ref, cos_ref, sin_ref, wq_ref, wk_ref, wv_ref,
              q_ref, k_ref, v_ref):
    xb = x_ref[...]
    xn = xb * jax.lax.rsqrt(jnp.mean(xb * xb, axis=1, keepdims=True) + EPS)
    xn = xn * n1_ref[...]
    q = _dot(xn, wq_ref[...])
    k = _dot(xn, wk_ref[...])
    v = _dot(xn, wv_ref[...])
    cos = cos_ref[...]
    sin = sin_ref[...]
    cosf = jnp.concatenate([cos] * nh, axis=1)
    sinf = jnp.concatenate([sin] * nh, axis=1)

    def rot(t):
        parts = []
        for h in range(nh):
            a = t[:, h * dh:h * dh + dh // 2]
            b = t[:, h * dh + dh // 2:(h + 1) * dh]
            parts.append(-b)
            parts.append(a)
        return jnp.concatenate(parts, axis=1)

    q_ref[...] = q * cosf + rot(q) * sinf
    k_ref[...] = k * cosf + rot(k) * sinf
    v_ref[...] = v


def _attn_body(bq, t_total, dh, scale, q_ref, k_ref, v_ref, o_ref):
    tq = pl.program_id(1)
    row = tq * bq + jax.lax.broadcasted_iota(jnp.int32, (bq, t_total), 0)
    col = jax.lax.broadcasted_iota(jnp.int32, (bq, t_total), 1)
    causal = row >= col
    for hh in range(q_ref.shape[1] // dh):
        sl = slice(hh * dh, (hh + 1) * dh)
        s = _dot(q_ref[:, sl], k_ref[:, sl]) * scale
        s = jnp.where(causal, s, NEG)
        m = jnp.max(s, axis=1, keepdims=True)
        p = jnp.exp(s - m)
        a = p / jnp.sum(p, axis=1, keepdims=True)
        o_ref[:, sl] = _dot(a, v_ref[:, sl], _C10)


def _post_body(ao_ref, x_ref, wo_ref, n2_ref, gw_ref, h_ref, xn2_ref, lgt_ref):
    hb = x_ref[...] + _dot(ao_ref[...], wo_ref[...])
    h_ref[...] = hb
    xn = hb * jax.lax.rsqrt(jnp.mean(hb * hb, axis=1, keepdims=True) + EPS)
    xn = xn * n2_ref[...]
    xn2_ref[...] = xn
    lgt_ref[...] = _dot(gw_ref[...], xn)


def _top2_weights(lg, axis, e_total):
    e_io = jax.lax.broadcasted_iota(jnp.int32, lg.shape, axis)
    m1 = jnp.max(lg, axis=axis, keepdims=True)
    e1 = jnp.min(jnp.where(lg == m1, e_io, e_total), axis=axis, keepdims=True)
    masked = jnp.where(e_io == e1, NEG, lg)
    m2 = jnp.max(masked, axis=axis, keepdims=True)
    e2 = jnp.min(jnp.where(masked == m2, e_io, e_total), axis=axis,
                 keepdims=True)
    r = jnp.exp(m2 - m1)
    w1 = 1.0 / (1.0 + r)
    w2 = r / (1.0 + r)
    w = jnp.where(e_io == e1, w1, 0.0) + jnp.where(e_io == e2, w2, 0.0)
    return w


def _route_body(n_tok, e_total, cap, jb, lgt_ref, wcap_ref):
    lgt = lgt_ref[...]
    lg = jax.lax.transpose(lgt, (1, 0))
    w_d = _top2_weights(lg, 1, e_total)
    w_t = _top2_weights(lgt, 0, e_total)
    row_io = jax.lax.broadcasted_iota(jnp.int32, (n_tok, 1), 0)
    ranks = []
    for e in range(e_total):
        mine = w_d[:, e:e + 1]
        acc = jnp.zeros((n_tok, 1), jnp.float32)
        for jc in range(n_tok // jb):
            oth = w_t[e:e + 1, jc * jb:(jc + 1) * jb]
            col_io = jc * jb + jax.lax.broadcasted_iota(
                jnp.int32, (1, jb), 1)
            ahead = (oth > mine) | ((oth == mine) & (col_io < row_io))
            acc = acc + jnp.sum(ahead.astype(jnp.float32), axis=1,
                                keepdims=True)
        ranks.append(acc)
    rank = jnp.concatenate(ranks, axis=1)
    sel = (w_d > 0.0) & (rank < cap)
    wcap = jnp.where(sel, w_d, 0.0)
    ones = jnp.ones((n_tok, 1), jnp.float32)
    wcap_ref[...] = jnp.concatenate([wcap, ones], axis=1)


def _moe_body(tb, n_tok, e_total, xn2_ref, h_ref, wcap_ref, wg_ref, wu_ref,
              wd_ref, out_ref):
    e = pl.program_id(0)
    ic = pl.program_id(1)
    wg = wg_ref[0]
    wu = wu_ref[0]
    wd = wd_ref[0]
    onehot = (jax.lax.broadcasted_iota(jnp.int32, (e_total + 1, 1), 0) == e
              ).astype(jnp.float32)
    first = (e == 0) & (ic == 0)
    for tc in range(n_tok // tb):
        sl = slice(tc * tb, (tc + 1) * tb)
        xc = xn2_ref[sl, :]
        g = _dot(xc, wg)
        u = _dot(xc, wu)
        t = g * jax.nn.sigmoid(g) * u
        y = _dot(t, wd)
        wcol = _dot(wcap_ref[sl, :], onehot, _C10)
        yw = y * wcol

        @pl.when(first)
        def _():
            out_ref[sl, :] = h_ref[sl, :] + yw

        @pl.when(jnp.logical_not(first))
        def _():
            out_ref[sl, :] = out_ref[sl, :] + yw


def kernel(x, freqs_cos, freqs_sin, norm1_w, norm2_w, Wq, Wk, Wv, Wo, gate_w,
           Weg, Weu, Wed, Wsg, Wsu, Wsd):
    B, T, H = x.shape
    DH = freqs_cos.shape[1]
    NH = H // DH
    E, I = Weg.shape[0], Weg.shape[1]
    N = B * T
    K = 2
    CAP = max(1, math.ceil(1.25 * N * K / E))

    xf = x.reshape(N, H)
    n1 = norm1_w.reshape(1, H)
    n2 = norm2_w.reshape(1, H)

    TB1 = 256
    grid1 = (N // TB1,)
    q, k, v = pl.pallas_call(
        lambda *refs: _qkv_body(NH, DH, *refs),
        grid=grid1,
        in_specs=[
            pl.BlockSpec((TB1, H), lambda i: (i, 0)),
            pl.BlockSpec((1, H), lambda i: (0, 0)),
            pl.BlockSpec((TB1, DH), lambda i: (i, 0)),
            pl.BlockSpec((TB1, DH), lambda i: (i, 0)),
            pl.BlockSpec((H, H), lambda i: (0, 0)),
            pl.BlockSpec((H, H), lambda i: (0, 0)),
            pl.BlockSpec((H, H), lambda i: (0, 0)),
        ],
        out_specs=[
            pl.BlockSpec((TB1, H), lambda i: (i, 0)),
            pl.BlockSpec((TB1, H), lambda i: (i, 0)),
            pl.BlockSpec((TB1, H), lambda i: (i, 0)),
        ],
        out_shape=[jax.ShapeDtypeStruct((N, H), jnp.float32)] * 3,
    )(xf, n1, freqs_cos, freqs_sin, Wq, Wk, Wv)

    BQ = 512
    HPB = 2
    scale = 1.0 / math.sqrt(DH)
    attn_out = pl.pallas_call(
        lambda *refs: _attn_body(BQ, T, DH, scale, *refs),
        grid=(NH // HPB, T // BQ),
        in_specs=[
            pl.BlockSpec((BQ, HPB * DH), lambda h, tq: (tq, h)),
            pl.BlockSpec((T, HPB * DH), lambda h, tq: (0, h)),
            pl.BlockSpec((T, HPB * DH), lambda h, tq: (0, h)),
        ],
        out_specs=pl.BlockSpec((BQ, HPB * DH), lambda h, tq: (tq, h)),
        out_shape=jax.ShapeDtypeStruct((N, H), jnp.float32),
    )(q, k, v)

    h_res, xn2, lgt = pl.pallas_call(
        _post_body,
        grid=grid1,
        in_specs=[
            pl.BlockSpec((TB1, H), lambda i: (i, 0)),
            pl.BlockSpec((TB1, H), lambda i: (i, 0)),
            pl.BlockSpec((H, H), lambda i: (0, 0)),
            pl.BlockSpec((1, H), lambda i: (0, 0)),
            pl.BlockSpec((E, H), lambda i: (0, 0)),
        ],
        out_specs=[
            pl.BlockSpec((TB1, H), lambda i: (i, 0)),
            pl.BlockSpec((TB1, H), lambda i: (i, 0)),
            pl.BlockSpec((E, TB1), lambda i: (0, i)),
        ],
        out_shape=[
            jax.ShapeDtypeStruct((N, H), jnp.float32),
            jax.ShapeDtypeStruct((N, H), jnp.float32),
            jax.ShapeDtypeStruct((E, N), jnp.float32),
        ],
    )(attn_out, xf, Wo, n2, gate_w)

    wcap = pl.pallas_call(
        lambda *refs: _route_body(N, E, CAP, 512, *refs),
        out_shape=jax.ShapeDtypeStruct((N, E + 1), jnp.float32),
    )(lgt)

    wg_all = jnp.concatenate([Weg, Wsg[None]], axis=0)
    wu_all = jnp.concatenate([Weu, Wsu[None]], axis=0)
    wd_all = jnp.concatenate([Wed, Wsd[None]], axis=0)

    IB = 512
    TBM = 256
    out = pl.pallas_call(
        lambda *refs: _moe_body(TBM, N, E, *refs),
        grid=(E + 1, I // IB),
        in_specs=[
            pl.BlockSpec((N, H), lambda e, ic: (0, 0)),
            pl.BlockSpec((N, H), lambda e, ic: (0, 0)),
            pl.BlockSpec((N, E + 1), lambda e, ic: (0, 0)),
            pl.BlockSpec((1, IB, H), lambda e, ic: (e, ic, 0)),
            pl.BlockSpec((1, IB, H), lambda e, ic: (e, ic, 0)),
            pl.BlockSpec((1, H, IB), lambda e, ic: (e, 0, ic)),
        ],
        out_specs=pl.BlockSpec((N, H), lambda e, ic: (0, 0)),
        out_shape=jax.ShapeDtypeStruct((N, H), jnp.float32),
    )(xn2, h_res, wcap, wg_all, wu_all, wd_all)

    return out.reshape(B, T, H), jnp.zeros(())
